# baseline (speedup 1.0000x reference)
import functools

import jax
import jax.numpy as jnp
from jax import lax
from jax.experimental import pallas as pl
from jax.experimental.pallas import tpu as pltpu
from jax.experimental.pallas import tpu_sc as plsc

D = 128
NC, NS = 2, 16
NW = NC * NS
B = 4096 * 200
BPW = B // NW
NUM_Q = 100000
SLAB = 10000
NPASS = NUM_Q // SLAB
FILL = 632
FILL_LAST = SLAB - 15 * FILL
GROUP = 64
NSLOT = 5
IDXROWS = 8
NIDXCH = BPW // (IDXROWS * 128)

_mesh = plsc.VectorSubcoreMesh(core_axis_name="c", subcore_axis_name="s")


@functools.partial(
    pl.kernel,
    out_type=jax.ShapeDtypeStruct((B + GROUP, D), jnp.float32),
    mesh=_mesh,
    compiler_params=pltpu.CompilerParams(needs_layout_passes=False),
    scratch_types=[
        pltpu.VMEM_SHARED((SLAB, D), jnp.float32),
        pltpu.VMEM((2, IDXROWS, 128), jnp.int32),
        pltpu.VMEM((96,), jnp.int32),
        pltpu.VMEM((96,), jnp.int32),
        pltpu.VMEM((NSLOT, GROUP), jnp.int32),
        pltpu.VMEM((NSLOT, GROUP), jnp.int32),
        pltpu.VMEM((NSLOT, GROUP, D), jnp.float32),
        pltpu.SemaphoreType.DMA,
        pltpu.SemaphoreType.DMA,
        pltpu.SemaphoreType.DMA,
        pltpu.SemaphoreType.DMA,
    ],
)
def _sc_gather(q_hbm, table_hbm, out_hbm, slab_v, ichunk_v, fidx_v, fpos_v,
               cidx_v, cpos_v, stage_v, i0, i1, gsem, osem):
    wid = lax.axis_index("s") * NC + lax.axis_index("c")
    tid = lax.axis_index("s")
    base = wid * BPW
    row0 = wid * (BPW // 128)
    isem = (i0, i1)

    def fire_ichunk(ci, buf):
        pltpu.async_copy(
            q_hbm.at[pl.ds(row0 + ci * IDXROWS, IDXROWS), :],
            ichunk_v.at[buf], isem[buf])

    def wait_ichunk(ci, buf):
        pltpu.make_async_copy(
            q_hbm.at[pl.ds(row0 + ci * IDXROWS, IDXROWS), :],
            ichunk_v.at[buf], isem[buf]).wait()

    def wait_one_gather():
        pltpu.make_async_copy(
            slab_v.at[cidx_v.at[0]], stage_v.at[0], gsem).wait()

    def wait_one_scatter():
        pltpu.make_async_copy(
            stage_v.at[0], out_hbm.at[cpos_v.at[0]], osem).wait()

    def fire_scatter(d):
        slot = lax.rem(d, NSLOT)
        pltpu.async_copy(
            stage_v.at[slot], out_hbm.at[cpos_v.at[slot]], osem)

    def drain(nacc, d, owaited):
        def free_slot(ow):
            wait_one_scatter()
            return ow + 1
        owaited = lax.cond(d >= NSLOT, free_slot, lambda ow: ow, owaited)

        slot = lax.rem(d, NSLOT)
        for t in range(GROUP // 16):
            cidx_v[slot, pl.ds(t * 16, 16)] = fidx_v[pl.ds(t * 16, 16)]
            cpos_v[slot, pl.ds(t * 16, 16)] = fpos_v[pl.ds(t * 16, 16)]
        fidx_v[pl.ds(0, 16)] = fidx_v[pl.ds(GROUP, 16)]
        fpos_v[pl.ds(0, 16)] = fpos_v[pl.ds(GROUP, 16)]

        pltpu.async_copy(slab_v.at[cidx_v.at[slot]], stage_v.at[slot], gsem)

        def emit_prev(_):
            wait_one_gather()
            fire_scatter(d - 1)
            return 0
        lax.cond(d >= 1, emit_prev, lambda _: 0, 0)
        return nacc - GROUP, d + 1, owaited

    def run_pass(s, carry0):
        lo = s * SLAB

        plsc.subcore_barrier()

        @pl.when(tid < 15)
        def _():
            pltpu.sync_copy(
                table_hbm.at[pl.ds(lo + tid * FILL, FILL), :],
                slab_v.at[pl.ds(tid * FILL, FILL), :])

        @pl.when(tid == 15)
        def _():
            pltpu.sync_copy(
                table_hbm.at[pl.ds(lo + 15 * FILL, FILL_LAST), :],
                slab_v.at[pl.ds(15 * FILL, FILL_LAST), :])

        plsc.subcore_barrier()

        fire_ichunk(0, 0)
        carry = (0, 0, 0)

        for ci in range(NIDXCH):
            buf = ci % 2
            wait_ichunk(ci, buf)
            if ci + 1 < NIDXCH:
                fire_ichunk(ci + 1, 1 - buf)

            def run_vreg(vi, carry, ci=ci, buf=buf):
                nacc, d, owaited = carry
                r = vi >> 3
                kk = vi & 7
                v = ichunk_v[buf, r, pl.ds(kk * 16, 16)]
                pos = (base + (ci * IDXROWS + r) * 128 + kk * 16
                       + lax.iota(jnp.int32, 16))
                m = (v >= lo) & (v < lo + SLAB)
                plsc.store_compressed(fidx_v.at[pl.ds(nacc, 16)],
                                      v - lo, mask=m)
                plsc.store_compressed(fpos_v.at[pl.ds(nacc, 16)],
                                      pos, mask=m)
                nacc = nacc + jnp.sum(m.astype(jnp.int32))
                return lax.cond(nacc >= GROUP, drain,
                                lambda a, b, c: (a, b, c),
                                nacc, d, owaited)

            carry = lax.fori_loop(0, IDXROWS * 8, run_vreg, carry)

        nacc, d, owaited = carry

        def flush(carry):
            nacc, d, owaited = carry
            for t in range(GROUP // 16):
                lane = t * 16 + lax.iota(jnp.int32, 16)
                valid = lane < nacc
                blk_i = fidx_v[pl.ds(t * 16, 16)]
                blk_p = fpos_v[pl.ds(t * 16, 16)]
                fidx_v[pl.ds(t * 16, 16)] = jnp.where(valid, blk_i, 0)
                fpos_v[pl.ds(t * 16, 16)] = jnp.where(valid, blk_p, B)
            return drain(nacc, d, owaited)

        nacc, d, owaited = lax.cond(
            nacc > 0, flush, lambda c: c, (nacc, d, owaited))

        def emit_last(_):
            wait_one_gather()
            fire_scatter(d - 1)
            return 0
        lax.cond(d >= 1, emit_last, lambda _: 0, 0)

        def wait_o(i, c):
            wait_one_scatter()
            return c
        lax.fori_loop(owaited, d, wait_o, 0)
        return carry0

    lax.fori_loop(0, NPASS, run_pass, 0)


def kernel(q, table):
    q_rows = q.reshape(B // 128, 128).astype(jnp.int32)
    out = _sc_gather(q_rows, table)
    return out[:B].reshape(q.shape[0], q.shape[1], D)


# speedup vs baseline: 4.5379x; 4.5379x over previous
import functools

import jax
import jax.numpy as jnp
from jax import lax
from jax.experimental import pallas as pl
from jax.experimental.pallas import tpu as pltpu
from jax.experimental.pallas import tpu_sc as plsc

D = 128
NC, NS = 2, 16
NW = NC * NS
B = 4096 * 200
BPW = B // NW
GSZ = 128
CH = 256
NSUB = CH // GSZ
NCHUNK = BPW // CH
NROWS_W = BPW // GSZ

_mesh = plsc.VectorSubcoreMesh(core_axis_name="c", subcore_axis_name="s")


@functools.partial(
    pl.kernel,
    out_type=jax.ShapeDtypeStruct((B, D), jnp.float32),
    mesh=_mesh,
    scratch_types=[
        pltpu.VMEM((NROWS_W, GSZ), jnp.int32),
        pltpu.VMEM((3, CH, D), jnp.float32),
        pltpu.SemaphoreType.DMA,
        pltpu.SemaphoreType.DMA,
        pltpu.SemaphoreType.DMA,
        pltpu.SemaphoreType.DMA,
        pltpu.SemaphoreType.DMA,
        pltpu.SemaphoreType.DMA,
    ],
)
def _sc_gather(q_hbm, table_hbm, out_hbm, idx_v, rows_v, g0, g1, g2, o0, o1, o2):
    wid = lax.axis_index("s") * NC + lax.axis_index("c")
    row0 = wid * NROWS_W
    base = wid * BPW
    gsem = (g0, g1, g2)
    osem = (o0, o1, o2)

    pltpu.sync_copy(q_hbm.at[pl.ds(row0, NROWS_W), :], idx_v)

    def fire_gathers(c, buf):
        for j in range(NSUB):
            pltpu.async_copy(
                table_hbm.at[idx_v.at[c * NSUB + j]],
                rows_v.at[buf, pl.ds(j * GSZ, GSZ), :],
                gsem[buf],
            )

    def wait_gathers(c, buf):
        for j in range(NSUB):
            pltpu.make_async_copy(
                table_hbm.at[idx_v.at[c * NSUB + j]],
                rows_v.at[buf, pl.ds(j * GSZ, GSZ), :],
                gsem[buf],
            ).wait()

    def fire_out(c, buf):
        pltpu.async_copy(
            rows_v.at[buf],
            out_hbm.at[pl.ds(base + c * CH, CH), :],
            osem[buf],
        )

    def wait_out(c, buf):
        pltpu.make_async_copy(
            rows_v.at[buf],
            out_hbm.at[pl.ds(base + c * CH, CH), :],
            osem[buf],
        ).wait()

    fire_gathers(0, 0)
    fire_gathers(1, 1)
    wait_gathers(0, 0)
    fire_out(0, 0)
    fire_gathers(2, 2)
    wait_gathers(1, 1)
    fire_out(1, 1)

    def step(c, buf):
        wait_gathers(c - 1, (buf + 2) % 3)
        fire_out(c - 1, (buf + 2) % 3)
        wait_out(c - 3, buf)
        fire_gathers(c, buf)

    def triple(p, carry):
        step(3 * p, 0)
        step(3 * p + 1, 1)
        step(3 * p + 2, 2)
        return carry

    lax.fori_loop(1, (NCHUNK - 1) // 3, triple, 0)

    step(NCHUNK - 1, 0)
    wait_gathers(NCHUNK - 1, 0)
    fire_out(NCHUNK - 1, 0)
    wait_out(NCHUNK - 3, 1)
    wait_out(NCHUNK - 2, 2)
    wait_out(NCHUNK - 1, 0)


def kernel(q, table):
    q_rows = q.reshape(B // GSZ, GSZ).astype(jnp.int32)
    out = _sc_gather(q_rows, table)
    return out.reshape(q.shape[0], q.shape[1], D)
